# Initial kernel scaffold; baseline (speedup 1.0000x reference)
import jax
import jax.numpy as jnp
from jax.experimental import pallas as pl


def kernel(x):
    raise NotImplementedError("write your pallas kernel here")


# speedup vs baseline: 1.8484x; 1.8484x over previous
import functools

import jax
import jax.numpy as jnp
from jax import lax
from jax.experimental import pallas as pl
from jax.experimental.pallas import tpu as pltpu
from jax.experimental.pallas import tpu_sc as plsc

PAD = 4
L = 16


def _make_crop_kernel(B, C, H, W):
    HW = H * W
    assert HW % L == 0
    n_vec = HW // L

    info = plsc.get_sparse_core_info()
    nw = info.num_cores * info.num_subcores
    assert B % nw == 0
    b_per_w = B // nw

    mesh = plsc.VectorSubcoreMesh(core_axis_name="c", subcore_axis_name="s")

    @functools.partial(
        pl.kernel,
        mesh=mesh,
        out_type=jax.ShapeDtypeStruct((B * C, HW), jnp.float32),
        compiler_params=pltpu.CompilerParams(needs_layout_passes=False),
        scratch_types=[
            pltpu.VMEM((B,), jnp.int32),
            pltpu.VMEM((B,), jnp.int32),
            pltpu.VMEM((HW,), jnp.int32),
            pltpu.VMEM((HW,), jnp.int32),
            pltpu.VMEM((HW,), jnp.int32),
            pltpu.VMEM((HW,), jnp.float32),
            pltpu.VMEM((HW,), jnp.float32),
        ],
    )
    def crop_kernel(x_hbm, dh_hbm, dw_hbm, hh_hbm, ww_hbm, out_hbm,
                    dh_v, dw_v, hh_v, ww_v, idx_v, xbuf, obuf):
        wid = lax.axis_index("s") * info.num_cores + lax.axis_index("c")

        pltpu.sync_copy(dh_hbm, dh_v)
        pltpu.sync_copy(dw_hbm, dw_v)
        pltpu.sync_copy(hh_hbm, hh_v)
        pltpu.sync_copy(ww_hbm, ww_v)

        def per_batch(i, carry):
            b = wid * b_per_w + i
            bvec = jnp.full((L,), b, dtype=jnp.int32)
            dhv = plsc.load_gather(dh_v, [bvec])
            dwv = plsc.load_gather(dw_v, [bvec])

            def idx_body(j, c2):
                s = pl.multiple_of(j * L, L)
                hh = hh_v[pl.ds(s, L)]
                ww = ww_v[pl.ds(s, L)]
                hs = jnp.clip(hh + dhv, 0, H - 1)
                ws = jnp.clip(ww + dwv, 0, W - 1)
                idx_v[pl.ds(s, L)] = hs * W + ws
                return c2

            lax.fori_loop(0, n_vec, idx_body, 0, unroll=4)

            def per_chan(c, c2):
                row = b * C + c
                pltpu.sync_copy(x_hbm.at[row], xbuf)

                def gather_body(j, c3):
                    s = pl.multiple_of(j * L, L)
                    ii = idx_v[pl.ds(s, L)]
                    obuf[pl.ds(s, L)] = plsc.load_gather(xbuf, [ii])
                    return c3

                lax.fori_loop(0, n_vec, gather_body, 0, unroll=4)
                pltpu.sync_copy(obuf, out_hbm.at[row])
                return c2

            lax.fori_loop(0, C, per_chan, 0)
            return carry

        lax.fori_loop(0, b_per_w, per_batch, 0)

    return crop_kernel


def kernel(x):
    B, C, H, W = x.shape

    k = jax.random.key(42)
    k1, k2 = jax.random.split(k)
    crop_h = jax.random.randint(k1, (B,), 0, 2 * PAD + 1)
    crop_w = jax.random.randint(k2, (B,), 0, 2 * PAD + 1)
    dh = (crop_h - PAD).astype(jnp.int32)
    dw = (crop_w - PAD).astype(jnp.int32)

    p = jnp.arange(H * W, dtype=jnp.int32)
    hh = p // W
    ww = p % W

    x2 = x.reshape(B * C, H * W)
    out = _make_crop_kernel(B, C, H, W)(x2, dh, dw, hh, ww)
    return out.reshape(B, C, H, W)


# speedup vs baseline: 2.1274x; 1.1509x over previous
import functools

import jax
import jax.numpy as jnp
from jax import lax
from jax.experimental import pallas as pl
from jax.experimental.pallas import tpu as pltpu
from jax.experimental.pallas import tpu_sc as plsc

PAD = 4
L = 16


def _make_crop_kernel(B, C, H, W):
    HW = H * W
    assert HW % L == 0
    n_vec = HW // L

    CHUNK = 3
    assert C % CHUNK == 0
    CW = CHUNK * HW
    chunks_per_b = C // CHUNK

    info = plsc.get_sparse_core_info()
    nw = info.num_cores * info.num_subcores
    assert B % nw == 0
    b_per_w = B // nw
    n_steps = b_per_w * chunks_per_b

    mesh = plsc.VectorSubcoreMesh(core_axis_name="c", subcore_axis_name="s")

    @functools.partial(
        pl.kernel,
        mesh=mesh,
        out_type=jax.ShapeDtypeStruct((B * chunks_per_b, CW), jnp.float32),
        compiler_params=pltpu.CompilerParams(needs_layout_passes=False),
        scratch_types=[
            pltpu.VMEM((B,), jnp.int32),
            pltpu.VMEM((B,), jnp.int32),
            pltpu.VMEM((HW,), jnp.int32),
            pltpu.VMEM((HW,), jnp.int32),
            pltpu.VMEM((HW,), jnp.int32),
            pltpu.VMEM((CW,), jnp.float32),
            pltpu.VMEM((CW,), jnp.float32),
            pltpu.VMEM((CW,), jnp.float32),
            pltpu.VMEM((CW,), jnp.float32),
            pltpu.SemaphoreType.DMA,
            pltpu.SemaphoreType.DMA,
            pltpu.SemaphoreType.DMA,
            pltpu.SemaphoreType.DMA,
        ],
    )
    def crop_kernel(x_hbm, dh_hbm, dw_hbm, hh_hbm, ww_hbm, out_hbm,
                    dh_v, dw_v, hh_v, ww_v, idx_v,
                    in0, in1, o0, o1, si0, si1, so0, so1):
        wid = lax.axis_index("s") * info.num_cores + lax.axis_index("c")
        base_row = wid * n_steps

        pltpu.sync_copy(dh_hbm, dh_v)
        pltpu.sync_copy(dw_hbm, dw_v)
        pltpu.sync_copy(hh_hbm, hh_v)
        pltpu.sync_copy(ww_hbm, ww_v)

        def in_copy(t, buf, sem):
            return pltpu.make_async_copy(x_hbm.at[base_row + t], buf, sem)

        def out_copy(t, buf, sem):
            return pltpu.make_async_copy(buf, out_hbm.at[base_row + t], sem)

        def compute_idx(b):
            bvec = jnp.full((L,), b, dtype=jnp.int32)
            dhv = plsc.load_gather(dh_v, [bvec])
            dwv = plsc.load_gather(dw_v, [bvec])

            def idx_body(j, c2):
                s = pl.multiple_of(j * L, L)
                hh = hh_v[pl.ds(s, L)]
                ww = ww_v[pl.ds(s, L)]
                hs = jnp.clip(hh + dhv, 0, H - 1)
                ws = jnp.clip(ww + dwv, 0, W - 1)
                idx_v[pl.ds(s, L)] = hs * W + ws
                return c2

            lax.fori_loop(0, n_vec, idx_body, 0, unroll=4)

        def gather_chunk(inbuf, obuf):
            for c in range(CHUNK):
                off = c * HW

                def g_body(j, c3, off=off):
                    s = pl.multiple_of(j * L, L)
                    ii = idx_v[pl.ds(s, L)]
                    if off:
                        ii = ii + jnp.int32(off)
                    obuf[pl.ds(off + s, L)] = plsc.load_gather(inbuf, [ii])
                    return c3

                lax.fori_loop(0, n_vec, g_body, 0, unroll=8)

        in_copy(0, in0, si0).start()

        def super_step(ti, carry):
            for par in range(2):
                t = ti * 2 + par
                inbuf, obuf, si, so = (
                    (in0, o0, si0, so0) if par == 0 else (in1, o1, si1, so1)
                )
                oin, osi = (in1, si1) if par == 0 else (in0, si0)

                @pl.when(t + 1 < n_steps)
                def _():
                    in_copy(t + 1, oin, osi).start()

                @pl.when(lax.rem(t, chunks_per_b) == 0)
                def _():
                    compute_idx(wid * b_per_w + lax.div(t, chunks_per_b))

                in_copy(t, inbuf, si).wait()

                @pl.when(t >= 2)
                def _():
                    out_copy(t - 2, obuf, so).wait()

                gather_chunk(inbuf, obuf)
                out_copy(t, obuf, so).start()
            return carry

        lax.fori_loop(0, n_steps // 2, super_step, 0)
        out_copy(n_steps - 2, o0, so0).wait()
        out_copy(n_steps - 1, o1, so1).wait()

    return crop_kernel


def kernel(x):
    B, C, H, W = x.shape

    k = jax.random.key(42)
    k1, k2 = jax.random.split(k)
    crop_h = jax.random.randint(k1, (B,), 0, 2 * PAD + 1)
    crop_w = jax.random.randint(k2, (B,), 0, 2 * PAD + 1)
    dh = (crop_h - PAD).astype(jnp.int32)
    dw = (crop_w - PAD).astype(jnp.int32)

    p = jnp.arange(H * W, dtype=jnp.int32)
    hh = p // W
    ww = p % W

    x2 = x.reshape(B * 3, 3 * H * W)
    out = _make_crop_kernel(B, C, H, W)(x2, dh, dw, hh, ww)
    return out.reshape(B, C, H, W)


# speedup vs baseline: 3.5686x; 1.6774x over previous
import functools

import jax
import jax.numpy as jnp
from jax import lax
from jax.experimental import pallas as pl
from jax.experimental.pallas import tpu as pltpu
from jax.experimental.pallas import tpu_sc as plsc

PAD = 4
L = 16


def _make_crop_kernel(B, C, H, W):
    HW = H * W
    assert HW % L == 0
    n_vec = HW // L

    CHUNK = 3
    assert C % CHUNK == 0
    CW = CHUNK * HW
    chunks_per_b = C // CHUNK

    info = plsc.get_sparse_core_info()
    nw = info.num_cores * info.num_subcores
    assert B % nw == 0
    b_per_w = B // nw
    n_steps = b_per_w * chunks_per_b

    mesh = plsc.VectorSubcoreMesh(core_axis_name="c", subcore_axis_name="s")

    @functools.partial(
        pl.kernel,
        mesh=mesh,
        out_type=jax.ShapeDtypeStruct((B * chunks_per_b, CW), jnp.float32),
        compiler_params=pltpu.CompilerParams(needs_layout_passes=False),
        scratch_types=[
            pltpu.VMEM((B,), jnp.int32),
            pltpu.VMEM((B,), jnp.int32),
            pltpu.VMEM((HW,), jnp.int32),
            pltpu.VMEM((HW,), jnp.int32),
            pltpu.VMEM((HW,), jnp.int32),
            pltpu.VMEM((CW,), jnp.float32),
            pltpu.VMEM((CW,), jnp.float32),
            pltpu.VMEM((CW,), jnp.float32),
            pltpu.VMEM((CW,), jnp.float32),
            pltpu.SemaphoreType.DMA,
            pltpu.SemaphoreType.DMA,
            pltpu.SemaphoreType.DMA,
            pltpu.SemaphoreType.DMA,
        ],
    )
    def crop_kernel(x_hbm, dh_hbm, dw_hbm, hh_hbm, ww_hbm, out_hbm,
                    dh_v, dw_v, hh_v, ww_v, idx_v,
                    in0, in1, o0, o1, si0, si1, so0, so1):
        wid = lax.axis_index("s") * info.num_cores + lax.axis_index("c")
        base_row = wid * n_steps

        pltpu.sync_copy(dh_hbm, dh_v)
        pltpu.sync_copy(dw_hbm, dw_v)
        pltpu.sync_copy(hh_hbm, hh_v)
        pltpu.sync_copy(ww_hbm, ww_v)

        def in_copy(t, buf, sem):
            return pltpu.make_async_copy(x_hbm.at[base_row + t], buf, sem)

        def out_copy(t, buf, sem):
            return pltpu.make_async_copy(buf, out_hbm.at[base_row + t], sem)

        def compute_idx(b):
            bvec = jnp.full((L,), b, dtype=jnp.int32)
            dhv = plsc.load_gather(dh_v, [bvec])
            dwv = plsc.load_gather(dw_v, [bvec])

            @plsc.parallel_loop(0, HW, L, unroll=4)
            def idx_body(s0):
                s = pl.multiple_of(s0, L)
                hh = hh_v[pl.ds(s, L)]
                ww = ww_v[pl.ds(s, L)]
                hs = jnp.clip(hh + dhv, 0, H - 1)
                ws = jnp.clip(ww + dwv, 0, W - 1)
                idx_v[pl.ds(s, L)] = hs * W + ws

        def gather_chunk(inbuf, obuf):
            for c in range(CHUNK):
                off = c * HW

                @plsc.parallel_loop(0, HW, L, unroll=8)
                def g_body(s0, off=off):
                    s = pl.multiple_of(s0, L)
                    ii = idx_v[pl.ds(s, L)]
                    if off:
                        ii = ii + jnp.int32(off)
                    obuf[pl.ds(off + s, L)] = plsc.load_gather(inbuf, [ii])

        in_copy(0, in0, si0).start()

        def super_step(ti, carry):
            for par in range(2):
                t = ti * 2 + par
                inbuf, obuf, si, so = (
                    (in0, o0, si0, so0) if par == 0 else (in1, o1, si1, so1)
                )
                oin, osi = (in1, si1) if par == 0 else (in0, si0)

                @pl.when(t + 1 < n_steps)
                def _():
                    in_copy(t + 1, oin, osi).start()

                @pl.when(lax.rem(t, chunks_per_b) == 0)
                def _():
                    compute_idx(wid * b_per_w + lax.div(t, chunks_per_b))

                in_copy(t, inbuf, si).wait()

                @pl.when(t >= 2)
                def _():
                    out_copy(t - 2, obuf, so).wait()

                gather_chunk(inbuf, obuf)
                out_copy(t, obuf, so).start()
            return carry

        lax.fori_loop(0, n_steps // 2, super_step, 0)
        out_copy(n_steps - 2, o0, so0).wait()
        out_copy(n_steps - 1, o1, so1).wait()

    return crop_kernel


def kernel(x):
    B, C, H, W = x.shape

    k = jax.random.key(42)
    k1, k2 = jax.random.split(k)
    crop_h = jax.random.randint(k1, (B,), 0, 2 * PAD + 1)
    crop_w = jax.random.randint(k2, (B,), 0, 2 * PAD + 1)
    dh = (crop_h - PAD).astype(jnp.int32)
    dw = (crop_w - PAD).astype(jnp.int32)

    p = jnp.arange(H * W, dtype=jnp.int32)
    hh = p // W
    ww = p % W

    x2 = x.reshape(B * 3, 3 * H * W)
    out = _make_crop_kernel(B, C, H, W)(x2, dh, dw, hh, ww)
    return out.reshape(B, C, H, W)


# speedup vs baseline: 4.6235x; 1.2956x over previous
import functools

import jax
import jax.numpy as jnp
from jax import lax
from jax.experimental import pallas as pl
from jax.experimental.pallas import tpu as pltpu
from jax.experimental.pallas import tpu_sc as plsc

PAD = 4
L = 16


def _make_crop_kernel(B, C, H, W):
    info = plsc.get_sparse_core_info()
    nw = info.num_cores * info.num_subcores
    assert B % nw == 0
    n_steps = (B // nw) * C

    w_starts = tuple(range(0, W - L, L)) + (W - L,)

    mesh = plsc.VectorSubcoreMesh(core_axis_name="c", subcore_axis_name="s")

    @functools.partial(
        pl.kernel,
        mesh=mesh,
        out_type=jax.ShapeDtypeStruct((B, C, H, W), jnp.float32),
        compiler_params=pltpu.CompilerParams(needs_layout_passes=False),
        scratch_types=[
            pltpu.VMEM((B,), jnp.int32),
            pltpu.VMEM((B,), jnp.int32),
            pltpu.VMEM((H, W), jnp.float32),
            pltpu.VMEM((H, W), jnp.float32),
            pltpu.VMEM((H, W), jnp.float32),
            pltpu.VMEM((H, W), jnp.float32),
            pltpu.SemaphoreType.DMA,
            pltpu.SemaphoreType.DMA,
            pltpu.SemaphoreType.DMA,
            pltpu.SemaphoreType.DMA,
        ],
    )
    def crop_kernel(x_hbm, dh_hbm, dw_hbm, out_hbm,
                    dh_v, dw_v, in0, in1, o0, o1, si0, si1, so0, so1):
        wid = lax.axis_index("s") * info.num_cores + lax.axis_index("c")
        base_t = wid * n_steps

        pltpu.sync_copy(dh_hbm, dh_v)
        pltpu.sync_copy(dw_hbm, dw_v)

        def bc(t):
            g = base_t + t
            return lax.div(g, C), lax.rem(g, C)

        def in_copy(t, buf, sem):
            b, c = bc(t)
            return pltpu.make_async_copy(x_hbm.at[b, c], buf, sem)

        def out_copy(t, buf, sem):
            b, c = bc(t)
            return pltpu.make_async_copy(buf, out_hbm.at[b, c], sem)

        iota = lax.iota(jnp.int32, L)

        def gather_image(t, inbuf, obuf):
            b, _ = bc(t)
            bvec = jnp.full((L,), b, dtype=jnp.int32)
            dhv = plsc.load_gather(dh_v, [bvec])
            dwv = plsc.load_gather(dw_v, [bvec])
            ws_list = [
                jnp.clip(iota + (dwv + w0), 0, W - 1) for w0 in w_starts
            ]

            @plsc.parallel_loop(0, H, 1, unroll=4)
            def row_body(h):
                hsv = jnp.clip(dhv + h, 0, H - 1)
                for w0, wsv in zip(w_starts, ws_list):
                    obuf[h, pl.ds(w0, L)] = plsc.load_gather(inbuf, [hsv, wsv])

        in_copy(0, in0, si0).start()

        def super_step(ti, carry):
            for par in range(2):
                t = ti * 2 + par
                inbuf, obuf, si, so = (
                    (in0, o0, si0, so0) if par == 0 else (in1, o1, si1, so1)
                )
                oin, osi = (in1, si1) if par == 0 else (in0, si0)

                @pl.when(t + 1 < n_steps)
                def _():
                    in_copy(t + 1, oin, osi).start()

                in_copy(t, inbuf, si).wait()

                @pl.when(t >= 2)
                def _():
                    out_copy(t - 2, obuf, so).wait()

                gather_image(t, inbuf, obuf)
                out_copy(t, obuf, so).start()
            return carry

        lax.fori_loop(0, n_steps // 2, super_step, 0)
        out_copy(n_steps - 2, o0, so0).wait()
        out_copy(n_steps - 1, o1, so1).wait()

    return crop_kernel


def kernel(x):
    B, C, H, W = x.shape

    k = jax.random.key(42)
    k1, k2 = jax.random.split(k)
    crop_h = jax.random.randint(k1, (B,), 0, 2 * PAD + 1)
    crop_w = jax.random.randint(k2, (B,), 0, 2 * PAD + 1)
    dh = (crop_h - PAD).astype(jnp.int32)
    dw = (crop_w - PAD).astype(jnp.int32)

    return _make_crop_kernel(B, C, H, W)(x, dh, dw)


# speedup vs baseline: 6.5669x; 1.4203x over previous
import functools

import jax
import jax.numpy as jnp
from jax import lax
from jax.experimental import pallas as pl
from jax.experimental.pallas import tpu as pltpu
from jax.experimental.pallas import tpu_sc as plsc
from jax.experimental import layout as jlayout

PAD = 4
L = 16


def _make_crop_kernel(B, C, H, W):
    info = plsc.get_sparse_core_info()
    nw = info.num_cores * info.num_subcores
    assert B % nw == 0
    n_steps = (B // nw) * C

    w_starts = tuple(range(0, W - L, L)) + (W - L,)

    mesh = plsc.VectorSubcoreMesh(core_axis_name="c", subcore_axis_name="s")

    @functools.partial(
        pl.kernel,
        mesh=mesh,
        out_type=jax.ShapeDtypeStruct((B, C, H, W), jnp.float32),
        compiler_params=pltpu.CompilerParams(needs_layout_passes=False),
        scratch_types=[
            pltpu.VMEM((B,), jnp.int32),
            pltpu.VMEM((B,), jnp.int32),
            pltpu.VMEM((H, W), jnp.float32),
            pltpu.VMEM((H, W), jnp.float32),
            pltpu.VMEM((H, W), jnp.float32),
            pltpu.VMEM((H, W), jnp.float32),
            pltpu.SemaphoreType.DMA,
            pltpu.SemaphoreType.DMA,
            pltpu.SemaphoreType.DMA,
            pltpu.SemaphoreType.DMA,
        ],
    )
    def crop_kernel(x_hbm, dh_hbm, dw_hbm, out_hbm,
                    dh_v, dw_v, in0, in1, o0, o1, si0, si1, so0, so1):
        wid = lax.axis_index("s") * info.num_cores + lax.axis_index("c")
        base_t = wid * n_steps

        pltpu.sync_copy(dh_hbm, dh_v)
        pltpu.sync_copy(dw_hbm, dw_v)

        def bc(t):
            g = base_t + t
            return lax.div(g, C), lax.rem(g, C)

        def in_copy(t, buf, sem):
            b, c = bc(t)
            return pltpu.make_async_copy(x_hbm.at[b, c], buf, sem)

        def out_copy(t, buf, sem):
            b, c = bc(t)
            return pltpu.make_async_copy(buf, out_hbm.at[b, c], sem)

        iota = lax.iota(jnp.int32, L)

        def gather_image(t, inbuf, obuf):
            b, _ = bc(t)
            bvec = jnp.full((L,), b, dtype=jnp.int32)
            dhv = plsc.load_gather(dh_v, [bvec])
            dwv = plsc.load_gather(dw_v, [bvec])
            ws_list = [
                jnp.clip(iota + (dwv + w0), 0, W - 1) for w0 in w_starts
            ]

            @plsc.parallel_loop(0, H, 1, unroll=4)
            def row_body(h):
                hsv = jnp.clip(dhv + h, 0, H - 1)
                for w0, wsv in zip(w_starts, ws_list):
                    obuf[h, pl.ds(w0, L)] = plsc.load_gather(inbuf, [hsv, wsv])

        in_copy(0, in0, si0).start()

        def super_step(ti, carry):
            for par in range(2):
                t = ti * 2 + par
                inbuf, obuf, si, so = (
                    (in0, o0, si0, so0) if par == 0 else (in1, o1, si1, so1)
                )
                oin, osi = (in1, si1) if par == 0 else (in0, si0)

                @pl.when(t + 1 < n_steps)
                def _():
                    in_copy(t + 1, oin, osi).start()

                in_copy(t, inbuf, si).wait()

                @pl.when(t >= 2)
                def _():
                    out_copy(t - 2, obuf, so).wait()

                gather_image(t, inbuf, obuf)
                out_copy(t, obuf, so).start()
            return carry

        lax.fori_loop(0, n_steps // 2, super_step, 0)
        out_copy(n_steps - 2, o0, so0).wait()
        out_copy(n_steps - 1, o1, so1).wait()

    return crop_kernel


def kernel(x):
    B, C, H, W = x.shape

    k = jax.random.key(42)
    k1, k2 = jax.random.split(k)
    crop_h = jax.random.randint(k1, (B,), 0, 2 * PAD + 1)
    crop_w = jax.random.randint(k2, (B,), 0, 2 * PAD + 1)
    dh = (crop_h - PAD).astype(jnp.int32)
    dw = (crop_w - PAD).astype(jnp.int32)

    out = _make_crop_kernel(B, C, H, W)(x, dh, dw)
    return jlayout.with_layout_constraint(
        out, jlayout.Layout(major_to_minor=(0, 1, 2, 3))
    )


# speedup vs baseline: 10.5113x; 1.6006x over previous
import functools

import jax
import jax.numpy as jnp
from jax import lax
from jax.experimental import pallas as pl
from jax.experimental.pallas import tpu as pltpu
from jax.experimental.pallas import tpu_sc as plsc
from jax.experimental import layout as jlayout

PAD = 4
L = 16
LANES = 128
RING = 2 * PAD + 2


def _make_crop_kernel(B, C, H, W):
    assert B % (2 * LANES) == 0
    n_sub = LANES // L

    n_planes = 2 * C * H

    info = plsc.get_sparse_core_info()
    nw = info.num_cores * info.num_subcores

    mesh = plsc.VectorSubcoreMesh(core_axis_name="c", subcore_axis_name="s")

    @functools.partial(
        pl.kernel,
        mesh=mesh,
        out_type=jax.ShapeDtypeStruct((C, H, W, B), jnp.float32),
        compiler_params=pltpu.CompilerParams(needs_layout_passes=False),
        scratch_types=[
            pltpu.VMEM((B,), jnp.int32),
            pltpu.VMEM((B,), jnp.int32),
            pltpu.VMEM((RING * W, LANES), jnp.float32),
            pltpu.VMEM((W, LANES), jnp.float32),
            pltpu.VMEM((W, LANES), jnp.float32),
            pltpu.SemaphoreType.DMA,
            pltpu.SemaphoreType.DMA,
            pltpu.SemaphoreType.DMA,
        ],
    )
    def crop_kernel(y_hbm, dh_hbm, dw_hbm, out_hbm,
                    dh_v, dw_v, ring_v, ob0, ob1, si, so0, so1):
        wid = lax.axis_index("s") * info.num_cores + lax.axis_index("c")
        q0 = lax.div(wid * n_planes, nw)
        q1 = lax.div((wid + 1) * n_planes, nw)

        pltpu.sync_copy(dh_hbm, dh_v)
        pltpu.sync_copy(dw_hbm, dw_v)

        iota = lax.iota(jnp.int32, L)

        def in_plane_copy(c, hsrc, half, slot):
            return pltpu.make_async_copy(
                y_hbm.at[c, hsrc, :, pl.ds(half * LANES, LANES)],
                ring_v.at[pl.ds(slot * W, W)], si)

        def out_plane_copy(c, h, half, obuf, so):
            return pltpu.make_async_copy(
                obuf,
                out_hbm.at[c, h, :, pl.ds(half * LANES, LANES)], so)

        def step(i, carry):
            pending, since = carry
            q = q0 + i
            half = lax.div(q, C * H)
            r = lax.rem(q, C * H)
            c = lax.div(r, H)
            h = lax.rem(r, H)

            refill = jnp.logical_or(h == 0, i == 0)

            @pl.when(jnp.logical_and(refill, pending >= 1))
            def _():
                in_plane_copy(0, 0, 0, 0).wait()

            @pl.when(jnp.logical_and(refill, pending >= 2))
            def _():
                in_plane_copy(0, 0, 0, 0).wait()

            @pl.when(refill)
            def _():
                for k in range(RING):
                    hp = h - PAD + k
                    slot = lax.rem(hp + RING, RING)
                    hs = jnp.clip(hp, 0, H - 1)
                    in_plane_copy(c, hs, half, slot).start()
                    in_plane_copy(c, hs, half, slot).wait()

            @pl.when(jnp.logical_and(~refill, since >= 2))
            def _():
                in_plane_copy(0, 0, 0, 0).wait()

            dhv, dwv, rbase, lnv = [], [], [], []
            for s in range(n_sub):
                bvec = iota + (half * LANES + s * L)
                dhv.append(plsc.load_gather(dh_v, [bvec]))
                dwv.append(plsc.load_gather(dw_v, [bvec]))
                rbase.append(
                    lax.rem(jnp.clip(dhv[s] + h, 0, H - 1), RING) * W)
                lnv.append(iota + (s * L))

            def do_plane(obuf, so, reuse_cond):
                @pl.when(reuse_cond)
                def _():
                    out_plane_copy(0, 0, 0, obuf, so).wait()

                @plsc.parallel_loop(0, W, 1, unroll=2)
                def w_body(wa):
                    for s in range(n_sub):
                        wsv = jnp.clip(dwv[s] + wa, 0, W - 1)
                        obuf[wa, pl.ds(s * L, L)] = plsc.load_gather(
                            ring_v, [rbase[s] + wsv, lnv[s]])

                out_plane_copy(c, h, half, obuf, so).start()

            even = lax.rem(i, 2) == 0

            @pl.when(even)
            def _():
                do_plane(ob0, so0, i >= 2)

            @pl.when(~even)
            def _():
                do_plane(ob1, so1, i >= 3)

            in_plane_copy(c, jnp.clip(h + RING - PAD, 0, H - 1), half,
                          lax.rem(h + RING - PAD, RING)).start()

            pending2 = jnp.where(
                refill, jnp.int32(1),
                jnp.where(since >= 2, pending, pending + 1))
            since2 = jnp.where(refill, jnp.int32(1), since + 1)
            return pending2, since2

        pending, _ = lax.fori_loop(
            0, q1 - q0, step, (jnp.int32(0), jnp.int32(0)))

        @pl.when(pending >= 1)
        def _():
            in_plane_copy(0, 0, 0, 0).wait()

        @pl.when(pending >= 2)
        def _():
            in_plane_copy(0, 0, 0, 0).wait()

        out_plane_copy(0, 0, 0, ob0, so0).wait()
        out_plane_copy(0, 0, 0, ob1, so1).wait()

    return crop_kernel


def kernel(x):
    B, C, H, W = x.shape

    k = jax.random.key(42)
    k1, k2 = jax.random.split(k)
    crop_h = jax.random.randint(k1, (B,), 0, 2 * PAD + 1)
    crop_w = jax.random.randint(k2, (B,), 0, 2 * PAD + 1)
    dh = (crop_h - PAD).astype(jnp.int32)
    dw = (crop_w - PAD).astype(jnp.int32)

    y = jnp.transpose(x, (1, 2, 3, 0))
    out_y = _make_crop_kernel(B, C, H, W)(y, dh, dw)
    out = jnp.transpose(out_y, (3, 0, 1, 2))
    return jlayout.with_layout_constraint(
        out, jlayout.Layout(major_to_minor=(1, 2, 3, 0))
    )


# speedup vs baseline: 11.0333x; 1.0497x over previous
import functools

import jax
import jax.numpy as jnp
from jax import lax
from jax.experimental import pallas as pl
from jax.experimental.pallas import tpu as pltpu
from jax.experimental.pallas import tpu_sc as plsc
from jax.experimental import layout as jlayout

PAD = 4
L = 16
LANES = 128
RING = 2 * PAD + 2


def _make_crop_kernel(B, C, H, W):
    assert B % (2 * LANES) == 0
    n_sub = LANES // L

    n_planes = 2 * C * H

    info = plsc.get_sparse_core_info()
    nw = info.num_cores * info.num_subcores

    mesh = plsc.VectorSubcoreMesh(core_axis_name="c", subcore_axis_name="s")

    @functools.partial(
        pl.kernel,
        mesh=mesh,
        out_type=jax.ShapeDtypeStruct((C, H, W, B), jnp.float32),
        compiler_params=pltpu.CompilerParams(needs_layout_passes=False),
        scratch_types=[
            pltpu.VMEM((B,), jnp.int32),
            pltpu.VMEM((B,), jnp.int32),
            pltpu.VMEM((RING * W, LANES), jnp.float32),
            pltpu.VMEM((W, LANES), jnp.float32),
            pltpu.VMEM((W, LANES), jnp.float32),
            pltpu.SemaphoreType.DMA,
            pltpu.SemaphoreType.DMA,
            pltpu.SemaphoreType.DMA,
        ],
    )
    def crop_kernel(y_hbm, dh_hbm, dw_hbm, out_hbm,
                    dh_v, dw_v, ring_v, ob0, ob1, si, so0, so1):
        wid = lax.axis_index("s") * info.num_cores + lax.axis_index("c")
        q0 = lax.div(wid * n_planes, nw)
        q1 = lax.div((wid + 1) * n_planes, nw)

        pltpu.sync_copy(dh_hbm, dh_v)
        pltpu.sync_copy(dw_hbm, dw_v)

        iota = lax.iota(jnp.int32, L)

        def in_plane_copy(c, hsrc, half, slot):
            return pltpu.make_async_copy(
                y_hbm.at[c, hsrc, :, pl.ds(half * LANES, LANES)],
                ring_v.at[pl.ds(slot * W, W)], si)

        def out_plane_copy(c, h, half, obuf, so):
            return pltpu.make_async_copy(
                obuf,
                out_hbm.at[c, h, :, pl.ds(half * LANES, LANES)], so)

        def step(i, carry):
            pending, since = carry
            q = q0 + i
            half = lax.div(q, C * H)
            r = lax.rem(q, C * H)
            c = lax.div(r, H)
            h = lax.rem(r, H)

            refill = jnp.logical_or(h == 0, i == 0)

            @pl.when(jnp.logical_and(refill, pending >= 1))
            def _():
                in_plane_copy(0, 0, 0, 0).wait()

            @pl.when(jnp.logical_and(refill, pending >= 2))
            def _():
                in_plane_copy(0, 0, 0, 0).wait()

            @pl.when(refill)
            def _():
                for k in range(RING):
                    hp = h - PAD + k
                    slot = lax.rem(hp + RING, RING)
                    hs = jnp.clip(hp, 0, H - 1)
                    in_plane_copy(c, hs, half, slot).start()
                    in_plane_copy(c, hs, half, slot).wait()

            @pl.when(jnp.logical_and(~refill, since >= 2))
            def _():
                in_plane_copy(0, 0, 0, 0).wait()

            dhv, dwv, rbase, rbd, lnv = [], [], [], [], []
            for s in range(n_sub):
                bvec = iota + (half * LANES + s * L)
                dhv.append(plsc.load_gather(dh_v, [bvec]))
                dwv.append(plsc.load_gather(dw_v, [bvec]))
                rbase.append(
                    lax.rem(jnp.clip(dhv[s] + h, 0, H - 1), RING) * W)
                rbd.append(rbase[s] + dwv[s])
                lnv.append(iota + (s * L))

            def do_plane(obuf, so, reuse_cond):
                @pl.when(reuse_cond)
                def _():
                    out_plane_copy(0, 0, 0, obuf, so).wait()

                for wa in range(PAD):
                    for s in range(n_sub):
                        wsv = jnp.maximum(dwv[s] + wa, 0)
                        obuf[wa, pl.ds(s * L, L)] = plsc.load_gather(
                            ring_v, [rbase[s] + wsv, lnv[s]])
                for wa in range(W - PAD, W):
                    for s in range(n_sub):
                        wsv = jnp.minimum(dwv[s] + wa, W - 1)
                        obuf[wa, pl.ds(s * L, L)] = plsc.load_gather(
                            ring_v, [rbase[s] + wsv, lnv[s]])

                @plsc.parallel_loop(PAD, W - PAD, 1, unroll=4)
                def w_body(wa):
                    for s in range(n_sub):
                        obuf[wa, pl.ds(s * L, L)] = plsc.load_gather(
                            ring_v, [rbd[s] + wa, lnv[s]])

                out_plane_copy(c, h, half, obuf, so).start()

            even = lax.rem(i, 2) == 0

            @pl.when(even)
            def _():
                do_plane(ob0, so0, i >= 2)

            @pl.when(~even)
            def _():
                do_plane(ob1, so1, i >= 3)

            in_plane_copy(c, jnp.clip(h + RING - PAD, 0, H - 1), half,
                          lax.rem(h + RING - PAD, RING)).start()

            pending2 = jnp.where(
                refill, jnp.int32(1),
                jnp.where(since >= 2, pending, pending + 1))
            since2 = jnp.where(refill, jnp.int32(1), since + 1)
            return pending2, since2

        pending, _ = lax.fori_loop(
            0, q1 - q0, step, (jnp.int32(0), jnp.int32(0)))

        @pl.when(pending >= 1)
        def _():
            in_plane_copy(0, 0, 0, 0).wait()

        @pl.when(pending >= 2)
        def _():
            in_plane_copy(0, 0, 0, 0).wait()

        out_plane_copy(0, 0, 0, ob0, so0).wait()
        out_plane_copy(0, 0, 0, ob1, so1).wait()

    return crop_kernel


def kernel(x):
    B, C, H, W = x.shape

    k = jax.random.key(42)
    k1, k2 = jax.random.split(k)
    crop_h = jax.random.randint(k1, (B,), 0, 2 * PAD + 1)
    crop_w = jax.random.randint(k2, (B,), 0, 2 * PAD + 1)
    dh = (crop_h - PAD).astype(jnp.int32)
    dw = (crop_w - PAD).astype(jnp.int32)

    y = jnp.transpose(x, (1, 2, 3, 0))
    out_y = _make_crop_kernel(B, C, H, W)(y, dh, dw)
    out = jnp.transpose(out_y, (3, 0, 1, 2))
    return jlayout.with_layout_constraint(
        out, jlayout.Layout(major_to_minor=(1, 2, 3, 0))
    )


# speedup vs baseline: 11.9009x; 1.0786x over previous
import functools

import jax
import jax.numpy as jnp
from jax import lax
from jax.experimental import pallas as pl
from jax.experimental.pallas import tpu as pltpu
from jax.experimental.pallas import tpu_sc as plsc
from jax.experimental import layout as jlayout

PAD = 4
L = 16
LANES = 128
RING = 2 * PAD + 2


def _make_crop_kernel(B, C, H, W):
    assert B % (2 * LANES) == 0
    n_sub = LANES // L

    n_planes = 2 * C * H

    info = plsc.get_sparse_core_info()
    nw = info.num_cores * info.num_subcores

    mesh = plsc.VectorSubcoreMesh(core_axis_name="c", subcore_axis_name="s")

    @functools.partial(
        pl.kernel,
        mesh=mesh,
        out_type=jax.ShapeDtypeStruct((C, H, W, B), jnp.float32),
        compiler_params=pltpu.CompilerParams(needs_layout_passes=False),
        scratch_types=[
            pltpu.VMEM((B,), jnp.int32),
            pltpu.VMEM((B,), jnp.int32),
            pltpu.VMEM((RING * W, LANES), jnp.float32),
            pltpu.VMEM((W, LANES), jnp.float32),
            pltpu.VMEM((W, LANES), jnp.float32),
            pltpu.SemaphoreType.DMA,
            pltpu.SemaphoreType.DMA,
            pltpu.SemaphoreType.DMA,
        ],
    )
    def crop_kernel(y_hbm, dh_hbm, dw_hbm, out_hbm,
                    dh_v, dw_v, ring_v, ob0, ob1, si, so0, so1):
        wid = lax.axis_index("s") * info.num_cores + lax.axis_index("c")
        q0 = lax.div(wid * n_planes, nw)
        q1 = lax.div((wid + 1) * n_planes, nw)

        pltpu.sync_copy(dh_hbm, dh_v)
        pltpu.sync_copy(dw_hbm, dw_v)

        iota = lax.iota(jnp.int32, L)

        def in_plane_copy(c, hsrc, half, slot):
            return pltpu.make_async_copy(
                y_hbm.at[c, hsrc, :, pl.ds(half * LANES, LANES)],
                ring_v.at[pl.ds(slot * W, W)], si)

        def out_plane_copy(c, h, half, obuf, so):
            return pltpu.make_async_copy(
                obuf,
                out_hbm.at[c, h, :, pl.ds(half * LANES, LANES)], so)

        def step(i, carry):
            pending, since = carry
            q = q0 + i
            half = lax.div(q, C * H)
            r = lax.rem(q, C * H)
            c = lax.div(r, H)
            h = lax.rem(r, H)

            refill = jnp.logical_or(h == 0, i == 0)

            @pl.when(jnp.logical_and(refill, pending >= 1))
            def _():
                in_plane_copy(0, 0, 0, 0).wait()

            @pl.when(jnp.logical_and(refill, pending >= 2))
            def _():
                in_plane_copy(0, 0, 0, 0).wait()

            @pl.when(refill)
            def _():
                for k in range(RING):
                    hp = h - PAD + k
                    slot = lax.rem(hp + RING, RING)
                    hs = jnp.clip(hp, 0, H - 1)
                    in_plane_copy(c, hs, half, slot).start()
                for _k in range(RING):
                    in_plane_copy(0, 0, 0, 0).wait()

            @pl.when(jnp.logical_and(~refill, since >= 2))
            def _():
                in_plane_copy(0, 0, 0, 0).wait()

            dhv, dwv, rbase, rbd, lnv = [], [], [], [], []
            for s in range(n_sub):
                bvec = iota + (half * LANES + s * L)
                dhv.append(plsc.load_gather(dh_v, [bvec]))
                dwv.append(plsc.load_gather(dw_v, [bvec]))
                rbase.append(
                    lax.rem(jnp.clip(dhv[s] + h, 0, H - 1), RING) * W)
                rbd.append(rbase[s] + dwv[s])
                lnv.append(iota + (s * L))

            def do_plane(obuf, so, reuse_cond):
                @pl.when(reuse_cond)
                def _():
                    out_plane_copy(0, 0, 0, obuf, so).wait()

                for wa in range(PAD):
                    for s in range(n_sub):
                        wsv = jnp.maximum(dwv[s] + wa, 0)
                        obuf[wa, pl.ds(s * L, L)] = plsc.load_gather(
                            ring_v, [rbase[s] + wsv, lnv[s]])
                for wa in range(W - PAD, W):
                    for s in range(n_sub):
                        wsv = jnp.minimum(dwv[s] + wa, W - 1)
                        obuf[wa, pl.ds(s * L, L)] = plsc.load_gather(
                            ring_v, [rbase[s] + wsv, lnv[s]])

                @plsc.parallel_loop(PAD, W - PAD, 1, unroll=8)
                def w_body(wa):
                    for s in range(n_sub):
                        obuf[wa, pl.ds(s * L, L)] = plsc.load_gather(
                            ring_v, [rbd[s] + wa, lnv[s]])

                out_plane_copy(c, h, half, obuf, so).start()

            even = lax.rem(i, 2) == 0

            @pl.when(even)
            def _():
                do_plane(ob0, so0, i >= 2)

            @pl.when(~even)
            def _():
                do_plane(ob1, so1, i >= 3)

            in_plane_copy(c, jnp.clip(h + RING - PAD, 0, H - 1), half,
                          lax.rem(h + RING - PAD, RING)).start()

            pending2 = jnp.where(
                refill, jnp.int32(1),
                jnp.where(since >= 2, pending, pending + 1))
            since2 = jnp.where(refill, jnp.int32(1), since + 1)
            return pending2, since2

        pending, _ = lax.fori_loop(
            0, q1 - q0, step, (jnp.int32(0), jnp.int32(0)))

        @pl.when(pending >= 1)
        def _():
            in_plane_copy(0, 0, 0, 0).wait()

        @pl.when(pending >= 2)
        def _():
            in_plane_copy(0, 0, 0, 0).wait()

        out_plane_copy(0, 0, 0, ob0, so0).wait()
        out_plane_copy(0, 0, 0, ob1, so1).wait()

    return crop_kernel


def kernel(x):
    B, C, H, W = x.shape

    k = jax.random.key(42)
    k1, k2 = jax.random.split(k)
    crop_h = jax.random.randint(k1, (B,), 0, 2 * PAD + 1)
    crop_w = jax.random.randint(k2, (B,), 0, 2 * PAD + 1)
    dh = (crop_h - PAD).astype(jnp.int32)
    dw = (crop_w - PAD).astype(jnp.int32)

    y = jnp.transpose(x, (1, 2, 3, 0))
    out_y = _make_crop_kernel(B, C, H, W)(y, dh, dw)
    out = jnp.transpose(out_y, (3, 0, 1, 2))
    return jlayout.with_layout_constraint(
        out, jlayout.Layout(major_to_minor=(1, 2, 3, 0))
    )


# speedup vs baseline: 11.9331x; 1.0027x over previous
import functools

import jax
import jax.numpy as jnp
from jax import lax
from jax.experimental import pallas as pl
from jax.experimental.pallas import tpu as pltpu
from jax.experimental.pallas import tpu_sc as plsc
from jax.experimental import layout as jlayout

PAD = 4
L = 16
LANES = 128
RING = 2 * PAD + 2


def _make_crop_kernel(B, C, H, W):
    assert B % (2 * LANES) == 0
    n_sub = LANES // L

    n_planes = 2 * C * H

    info = plsc.get_sparse_core_info()
    nw = info.num_cores * info.num_subcores

    mesh = plsc.VectorSubcoreMesh(core_axis_name="c", subcore_axis_name="s")

    @functools.partial(
        pl.kernel,
        mesh=mesh,
        out_type=jax.ShapeDtypeStruct((C, H, W, B), jnp.float32),
        compiler_params=pltpu.CompilerParams(
            needs_layout_passes=False, skip_device_barrier=True),
        scratch_types=[
            pltpu.VMEM((B,), jnp.int32),
            pltpu.VMEM((B,), jnp.int32),
            pltpu.VMEM((RING * W, LANES), jnp.float32),
            pltpu.VMEM((W, LANES), jnp.float32),
            pltpu.VMEM((W, LANES), jnp.float32),
            pltpu.SemaphoreType.DMA,
            pltpu.SemaphoreType.DMA,
            pltpu.SemaphoreType.DMA,
        ],
    )
    def crop_kernel(y_hbm, dh_hbm, dw_hbm, out_hbm,
                    dh_v, dw_v, ring_v, ob0, ob1, si, so0, so1):
        wid = lax.axis_index("s") * info.num_cores + lax.axis_index("c")
        q0 = lax.div(wid * n_planes, nw)
        q1 = lax.div((wid + 1) * n_planes, nw)

        pltpu.sync_copy(dh_hbm, dh_v)
        pltpu.sync_copy(dw_hbm, dw_v)

        iota = lax.iota(jnp.int32, L)

        def in_plane_copy(c, hsrc, half, slot):
            return pltpu.make_async_copy(
                y_hbm.at[c, hsrc, :, pl.ds(half * LANES, LANES)],
                ring_v.at[pl.ds(slot * W, W)], si)

        def out_plane_copy(c, h, half, obuf, so):
            return pltpu.make_async_copy(
                obuf,
                out_hbm.at[c, h, :, pl.ds(half * LANES, LANES)], so)

        def step(i, carry):
            pending, since = carry
            q = q0 + i
            half = lax.div(q, C * H)
            r = lax.rem(q, C * H)
            c = lax.div(r, H)
            h = lax.rem(r, H)

            refill = jnp.logical_or(h == 0, i == 0)

            @pl.when(jnp.logical_and(refill, pending >= 1))
            def _():
                in_plane_copy(0, 0, 0, 0).wait()

            @pl.when(jnp.logical_and(refill, pending >= 2))
            def _():
                in_plane_copy(0, 0, 0, 0).wait()

            @pl.when(refill)
            def _():
                for k in range(RING):
                    hp = h - PAD + k
                    slot = lax.rem(hp + RING, RING)
                    hs = jnp.clip(hp, 0, H - 1)
                    in_plane_copy(c, hs, half, slot).start()
                for _k in range(RING):
                    in_plane_copy(0, 0, 0, 0).wait()

            @pl.when(jnp.logical_and(~refill, since >= 2))
            def _():
                in_plane_copy(0, 0, 0, 0).wait()

            dhv, dwv, rbase, rbd, lnv = [], [], [], [], []
            for s in range(n_sub):
                bvec = iota + (half * LANES + s * L)
                dhv.append(plsc.load_gather(dh_v, [bvec]))
                dwv.append(plsc.load_gather(dw_v, [bvec]))
                rbase.append(
                    lax.rem(jnp.clip(dhv[s] + h, 0, H - 1), RING) * W)
                rbd.append(rbase[s] + dwv[s])
                lnv.append(iota + (s * L))

            def do_plane(obuf, so, reuse_cond):
                @pl.when(reuse_cond)
                def _():
                    out_plane_copy(0, 0, 0, obuf, so).wait()

                for wa in range(PAD):
                    for s in range(n_sub):
                        wsv = jnp.maximum(dwv[s] + wa, 0)
                        obuf[wa, pl.ds(s * L, L)] = plsc.load_gather(
                            ring_v, [rbase[s] + wsv, lnv[s]])
                for wa in range(W - PAD, W):
                    for s in range(n_sub):
                        wsv = jnp.minimum(dwv[s] + wa, W - 1)
                        obuf[wa, pl.ds(s * L, L)] = plsc.load_gather(
                            ring_v, [rbase[s] + wsv, lnv[s]])

                @plsc.parallel_loop(PAD, W - PAD, 1, unroll=8)
                def w_body(wa):
                    for s in range(n_sub):
                        obuf[wa, pl.ds(s * L, L)] = plsc.load_gather(
                            ring_v, [rbd[s] + wa, lnv[s]])

                out_plane_copy(c, h, half, obuf, so).start()

            even = lax.rem(i, 2) == 0

            @pl.when(even)
            def _():
                do_plane(ob0, so0, i >= 2)

            @pl.when(~even)
            def _():
                do_plane(ob1, so1, i >= 3)

            in_plane_copy(c, jnp.clip(h + RING - PAD, 0, H - 1), half,
                          lax.rem(h + RING - PAD, RING)).start()

            pending2 = jnp.where(
                refill, jnp.int32(1),
                jnp.where(since >= 2, pending, pending + 1))
            since2 = jnp.where(refill, jnp.int32(1), since + 1)
            return pending2, since2

        pending, _ = lax.fori_loop(
            0, q1 - q0, step, (jnp.int32(0), jnp.int32(0)))

        @pl.when(pending >= 1)
        def _():
            in_plane_copy(0, 0, 0, 0).wait()

        @pl.when(pending >= 2)
        def _():
            in_plane_copy(0, 0, 0, 0).wait()

        out_plane_copy(0, 0, 0, ob0, so0).wait()
        out_plane_copy(0, 0, 0, ob1, so1).wait()

    return crop_kernel


def kernel(x):
    B, C, H, W = x.shape

    k = jax.random.key(42)
    k1, k2 = jax.random.split(k)
    crop_h = jax.random.randint(k1, (B,), 0, 2 * PAD + 1)
    crop_w = jax.random.randint(k2, (B,), 0, 2 * PAD + 1)
    dh = (crop_h - PAD).astype(jnp.int32)
    dw = (crop_w - PAD).astype(jnp.int32)

    y = jnp.transpose(x, (1, 2, 3, 0))
    out_y = _make_crop_kernel(B, C, H, W)(y, dh, dw)
    out = jnp.transpose(out_y, (3, 0, 1, 2))
    return jlayout.with_layout_constraint(
        out, jlayout.Layout(major_to_minor=(1, 2, 3, 0))
    )
